# baseline (speedup 1.0000x reference)
import math

import jax
import jax.numpy as jnp
from jax.experimental import pallas as pl
from jax.experimental.pallas import tpu as pltpu

_B, _C, _N = 1024, 1024, 16
_NLEAF = _N * _N * _N
_CH = 256
_NCH = _NLEAF // _CH
_GRP = _CH // _N
_INV_SQRT2 = 1.0 / math.sqrt(2.0)


def _res_scale(level):
    theta = math.pi / 3.0 / (1.5 ** level)
    return math.sqrt(1.0 / _C * (1.0 / math.cos(theta) ** 2 - 1.0))


_C1 = _res_scale(1)
_C2 = _res_scale(2)


def _nrm(a):
    n = jnp.sqrt(jnp.sum(a * a, axis=-1, keepdims=True))
    return a / jnp.maximum(n, 1e-12)


def _top2(v):
    iota = jax.lax.broadcasted_iota(jnp.int32, v.shape, 0)
    vmax = jnp.max(v, axis=0, keepdims=True)
    i1 = jnp.min(jnp.where(v == vmax, iota, _N), axis=0, keepdims=True)
    masked = jnp.where(iota == i1, -jnp.inf, v)
    vmax2 = jnp.max(masked, axis=0, keepdims=True)
    i2 = jnp.min(jnp.where(masked == vmax2, iota, _N), axis=0, keepdims=True)
    return i1, i2


def _hnsw_kernel(x_ref, ns_ref, gw0_ref, gw1_ref, m1_ref, m2_ref, sc_ref,
                 out_ref, t_ref, g1n_ref, pairs_ref, g_ref):
    lc = pl.program_id(0)

    @pl.when(lc == 0)
    def _route():
        x = x_ref[...]
        t = x * jax.lax.rsqrt(jnp.mean(x * x, axis=-1, keepdims=True) + 1e-5)
        t = t * ns_ref[...]
        t_ref[...] = t
        g0 = _nrm(gw0_ref[...])
        g0r = _nrm(g0)
        g1 = _nrm(jnp.reshape(gw1_ref[...], (_N, _N, _C)) * _C1
                  + g0r[:, None, :])
        g1 = jnp.reshape(g1, (_N * _N, _C))
        g1n_ref[...] = _nrm(g1)
        s0 = jax.lax.dot_general(g0, t, (((1,), (1,)), ((), ())),
                                 preferred_element_type=jnp.float32)
        S1 = jax.lax.dot_general(g1, t, (((1,), (1,)), ((), ())),
                                 preferred_element_type=jnp.float32)
        i0a, i0b = _top2(s0)
        rows = []
        for i0 in (i0a, i0b):
            s1 = jnp.zeros((_N, _B), jnp.float32)
            for m in range(_N):
                s1 = s1 + jnp.where(i0 == m, S1[m * _N:(m + 1) * _N, :], 0.0)
            j1, j2 = _top2(s1)
            rows.append(i0 * _N + j1)
            rows.append(i0 * _N + j2)
        pairs_ref[...] = jnp.concatenate(rows, axis=0)

    @pl.when(lc < _NCH)
    def _chunk():
        par = g1n_ref[pl.ds(lc * _GRP, _GRP), :]
        k = _nrm(jnp.reshape(m1_ref[...], (_GRP, _N, _C)) * _C2
                 + par[:, None, :])
        k = jnp.reshape(k, (_CH, _C))
        t = t_ref[...]
        hk = jax.lax.dot_general(k, t, (((1,), (1,)), ((), ())),
                                 preferred_element_type=jnp.float32)
        codeg = lc * _GRP + jax.lax.broadcasted_iota(jnp.int32, (_GRP, _B), 0)
        pr = pairs_ref[...]
        mg = ((codeg == pr[0:1, :]) | (codeg == pr[1:2, :])
              | (codeg == pr[2:3, :]) | (codeg == pr[3:4, :]))
        mask = jnp.reshape(jnp.broadcast_to(mg[:, None, :], (_GRP, _N, _B)),
                           (_CH, _B))
        z = sc_ref[0, 0] * hk
        act = 0.5 * z * (1.0 + jax.lax.erf(z * _INV_SQRT2))
        g_ref[pl.ds(lc * _CH, _CH), :] = jnp.where(mask, act, 0.0)

    @pl.when(lc == _NCH)
    def _combine():
        contrib = jax.lax.dot_general(g_ref[...], m2_ref[...],
                                      (((0,), (0,)), ((), ())),
                                      preferred_element_type=jnp.float32)
        out_ref[...] = x_ref[...] + contrib


def kernel(x, norm_scale, gate_w0, gate_w1, mlp1_w, mlp2_w, scale):
    gw1 = jnp.reshape(gate_w1, (_N * _N, _C))
    m1 = jnp.reshape(mlp1_w, (_NLEAF, _C))
    m2 = jnp.reshape(mlp2_w, (_NLEAF, _C)).astype(jnp.bfloat16)
    ns = jnp.reshape(norm_scale, (1, _C))
    sc = jnp.reshape(scale, (1, 1))
    return pl.pallas_call(
        _hnsw_kernel,
        grid=(_NCH + 1,),
        in_specs=[
            pl.BlockSpec((_B, _C), lambda lc: (0, 0)),
            pl.BlockSpec((1, _C), lambda lc: (0, 0)),
            pl.BlockSpec((_N, _C), lambda lc: (0, 0)),
            pl.BlockSpec((_N * _N, _C), lambda lc: (0, 0)),
            pl.BlockSpec((_CH, _C),
                         lambda lc: (jnp.minimum(lc, _NCH - 1), 0)),
            pl.BlockSpec((_NLEAF, _C), lambda lc: (0, 0)),
            pl.BlockSpec(memory_space=pltpu.SMEM),
        ],
        out_specs=pl.BlockSpec((_B, _C), lambda lc: (0, 0)),
        out_shape=jax.ShapeDtypeStruct((_B, _C), jnp.float32),
        scratch_shapes=[
            pltpu.VMEM((_B, _C), jnp.float32),
            pltpu.VMEM((_N * _N, _C), jnp.float32),
            pltpu.VMEM((4, _B), jnp.int32),
            pltpu.VMEM((_NLEAF, _B), jnp.float32),
        ],
        compiler_params=pltpu.CompilerParams(
            dimension_semantics=("arbitrary",),
        ),
    )(x, ns, gate_w0, gw1, m1, m2, sc)


# speedup vs baseline: 1.3019x; 1.3019x over previous
import math

import jax
import jax.numpy as jnp
from jax.experimental import pallas as pl
from jax.experimental.pallas import tpu as pltpu

_B, _C, _N = 1024, 1024, 16
_NLEAF = _N * _N * _N
_CH = 512
_NCH = _NLEAF // _CH
_GRP = _CH // _N
_INV_SQRT2 = 1.0 / math.sqrt(2.0)


def _res_scale(level):
    theta = math.pi / 3.0 / (1.5 ** level)
    return math.sqrt(1.0 / _C * (1.0 / math.cos(theta) ** 2 - 1.0))


_C1 = _res_scale(1)
_C2 = _res_scale(2)


def _nrm(a):
    n = jnp.sqrt(jnp.sum(a * a, axis=-1, keepdims=True))
    return a / jnp.maximum(n, 1e-12)


def _top2(v):
    iota = jax.lax.broadcasted_iota(jnp.int32, v.shape, 0)
    vmax = jnp.max(v, axis=0, keepdims=True)
    i1 = jnp.min(jnp.where(v == vmax, iota, _N), axis=0, keepdims=True)
    masked = jnp.where(iota == i1, -jnp.inf, v)
    vmax2 = jnp.max(masked, axis=0, keepdims=True)
    i2 = jnp.min(jnp.where(masked == vmax2, iota, _N), axis=0, keepdims=True)
    return i1, i2


def _hnsw_kernel(x_ref, ns_ref, gw0_ref, gw1_ref, m1_ref, m2_ref, sc_ref,
                 out_ref, t_ref, g1n_ref, pairs_ref):
    lc = pl.program_id(0)

    @pl.when(lc == 0)
    def _route():
        x = x_ref[...]
        t = x * jax.lax.rsqrt(jnp.mean(x * x, axis=-1, keepdims=True) + 1e-5)
        t = t * ns_ref[...]
        t_ref[...] = t
        g0 = _nrm(gw0_ref[...])
        g0r = _nrm(g0)
        g1 = _nrm(jnp.reshape(gw1_ref[...], (_N, _N, _C)) * _C1
                  + g0r[:, None, :])
        g1 = jnp.reshape(g1, (_N * _N, _C))
        g1n_ref[...] = _nrm(g1)
        s0 = jax.lax.dot_general(g0, t, (((1,), (1,)), ((), ())),
                                 preferred_element_type=jnp.float32)
        S1 = jax.lax.dot_general(g1, t, (((1,), (1,)), ((), ())),
                                 preferred_element_type=jnp.float32)
        i0a, i0b = _top2(s0)
        rows = []
        for i0 in (i0a, i0b):
            s1 = jnp.zeros((_N, _B), jnp.float32)
            for m in range(_N):
                s1 = s1 + jnp.where(i0 == m, S1[m * _N:(m + 1) * _N, :], 0.0)
            j1, j2 = _top2(s1)
            rows.append(i0 * _N + j1)
            rows.append(i0 * _N + j2)
        pairs_ref[...] = jnp.concatenate(rows, axis=0)

    par = g1n_ref[pl.ds(lc * _GRP, _GRP), :]
    k = _nrm(jnp.reshape(m1_ref[...], (_GRP, _N, _C)) * _C2
             + par[:, None, :])
    k = jnp.reshape(k, (_CH, _C))
    t = t_ref[...]
    hk = jax.lax.dot_general(k, t, (((1,), (1,)), ((), ())),
                             preferred_element_type=jnp.float32)
    codeg = lc * _GRP + jax.lax.broadcasted_iota(jnp.int32, (_GRP, _B), 0)
    pr = pairs_ref[...]
    mg = ((codeg == pr[0:1, :]) | (codeg == pr[1:2, :])
          | (codeg == pr[2:3, :]) | (codeg == pr[3:4, :]))
    mask = jnp.reshape(jnp.broadcast_to(mg[:, None, :], (_GRP, _N, _B)),
                       (_CH, _B))
    z = sc_ref[0, 0] * hk
    act = 0.5 * z * (1.0 + jax.lax.erf(z * _INV_SQRT2))
    g = jnp.where(mask, act, 0.0)
    contrib = jax.lax.dot_general(g, m2_ref[...], (((0,), (0,)), ((), ())),
                                  preferred_element_type=jnp.float32)

    @pl.when(lc == 0)
    def _first():
        out_ref[...] = x_ref[...] + contrib

    @pl.when(lc != 0)
    def _rest():
        out_ref[...] += contrib


def kernel(x, norm_scale, gate_w0, gate_w1, mlp1_w, mlp2_w, scale):
    gw1 = jnp.reshape(gate_w1, (_N * _N, _C))
    m1 = jnp.reshape(mlp1_w, (_NLEAF, _C))
    m2 = jnp.reshape(mlp2_w, (_NLEAF, _C))
    ns = jnp.reshape(norm_scale, (1, _C))
    sc = jnp.reshape(scale, (1, 1))
    return pl.pallas_call(
        _hnsw_kernel,
        grid=(_NCH,),
        in_specs=[
            pl.BlockSpec((_B, _C), lambda lc: (0, 0)),
            pl.BlockSpec((1, _C), lambda lc: (0, 0)),
            pl.BlockSpec((_N, _C), lambda lc: (0, 0)),
            pl.BlockSpec((_N * _N, _C), lambda lc: (0, 0)),
            pl.BlockSpec((_CH, _C), lambda lc: (lc, 0)),
            pl.BlockSpec((_CH, _C), lambda lc: (lc, 0)),
            pl.BlockSpec((1, 1), lambda lc: (0, 0)),
        ],
        out_specs=pl.BlockSpec((_B, _C), lambda lc: (0, 0)),
        out_shape=jax.ShapeDtypeStruct((_B, _C), jnp.float32),
        scratch_shapes=[
            pltpu.VMEM((_B, _C), jnp.float32),
            pltpu.VMEM((_N * _N, _C), jnp.float32),
            pltpu.VMEM((4, _B), jnp.int32),
        ],
        compiler_params=pltpu.CompilerParams(
            dimension_semantics=("arbitrary",),
        ),
    )(x, ns, gate_w0, gw1, m1, m2, sc)
